# baseline (speedup 1.0000x reference)
import functools

import jax
import jax.numpy as jnp
from jax import lax
from jax.experimental import pallas as pl
from jax.experimental.pallas import tpu as pltpu
from jax.experimental.pallas import tpu_sc as plsc

N = 10000
E = 320000
D = 128
K = 25
KS = 5
KMAX = 4

NC = 2
NS = 16
NW = NC * NS
E_PAD = 327680

B1 = 64
NSL = 2
NPAIR = 2
TROW = 30
TCOL = 2 * D
EPW = E_PAD // NW
C1 = EPW // B1

B2 = 128
EPC = E_PAD // NS
C2 = EPC // B2

RSZ = 2560
NRANGE = 4
ACC_ROWS = RSZ + 128
ZPS = ACC_ROWS // NS
CPS = RSZ // NS
N_SH = 10112


def _xw_body(x_ref, w_ref, o_ref):
    o_ref[...] = jnp.dot(x_ref[...], w_ref[...], preferred_element_type=jnp.float32)


def _finalize_body(p_ref, dp_ref, x_ref, wr_ref, b_ref, o_ref):
    deg = jnp.sum(dp_ref[...], axis=0)[:, None]
    root = jnp.dot(x_ref[...], wr_ref[...], preferred_element_type=jnp.float32)
    o_ref[...] = p_ref[...] / jnp.maximum(deg, 1.0) + root + b_ref[...]


def _sc_msg_body(ed_hbm, pp_hbm, xw_hbm, msg_hbm, deg_hbm,
                 ed0, ed1, pp0, pp1,
                 gi0, gi1, rows0, rows1,
                 acc0, acc1, deg_v,
                 sin0, sin1, sp0, sp1,
                 sg0, sg1, sw0, sw1):
    cid = lax.axis_index("c")
    sid = lax.axis_index("s")
    wid = sid * NC + cid
    cbase = wid * C1

    ed = (ed0, ed1)
    pp = (pp0, pp1)
    sp = (sp0, sp1)
    gi = (gi0, gi1)
    rows = (rows0, rows1)
    acc = (acc0, acc1)
    sin = (sin0, sin1)
    sg = (sg0, sg1)
    sw = (sw0, sw1)

    onehot = jnp.where(lax.iota(jnp.int32, 16) == 0, 1.0, 0.0).astype(jnp.float32)
    zeros16 = jnp.zeros((16,), jnp.float32)

    def _zero_deg(i, _):
        deg_v[pl.ds(i * 16, 16)] = zeros16
        return ()
    lax.fori_loop(0, (N_SH + 16) // 16, _zero_deg, ())

    def fire_ed(c, s):
        pltpu.async_copy(ed_hbm.at[cbase + c], ed[s], sin[s])
        pltpu.async_copy(pp_hbm.at[cbase + c], pp[s], sp[s])

    def wait_ed(c, s):
        pltpu.make_async_copy(ed_hbm.at[cbase + c], ed[s], sin[s]).wait()
        pltpu.make_async_copy(pp_hbm.at[cbase + c], pp[s], sp[s]).wait()

    def phase_a(s):
        edv = ed[s]
        ppv = pp[s]
        giv = gi[s]
        for g in range(B1 // 16):
            sl = pl.ds(g * 16, 16)
            srcv = edv[1, sl]
            p0 = ppv[0, sl]
            p1 = ppv[1, sl]
            b0i = (p0 * float(KS - 1)).astype(jnp.int32)
            b1i = (p1 * float(KS - 1)).astype(jnp.int32)
            g0 = srcv * TROW + b1i * KS + b0i
            giv[0, sl] = g0
            giv[1, sl] = g0 + KS

    def fire_gathers(s):
        for t in range(NPAIR):
            pltpu.async_copy(xw_hbm.at[gi[s].at[t]], rows[s].at[t], sg[s])

    def wait_gathers(s):
        for t in range(NPAIR):
            pltpu.make_async_copy(xw_hbm.at[gi[s].at[t]], rows[s].at[t],
                                  sg[s]).wait()

    def fire_write(c, s):
        pltpu.async_copy(acc[s], msg_hbm.at[pl.ds((cbase + c) * B1, B1)], sw[s])

    def wait_write(c, s):
        pltpu.make_async_copy(acc[s], msg_hbm.at[pl.ds((cbase + c) * B1, B1)],
                              sw[s]).wait()

    def phase_b(s):
        edv = ed[s]
        ppv = pp[s]
        rv = rows[s]
        av = acc[s]

        def _group(g, _):
            sl16 = pl.ds(g * 16, 16)
            dst16 = edv[0, sl16]
            p0 = ppv[0, sl16]
            p1 = ppv[1, sl16]
            v0 = p0 * float(KS - 1)
            v1 = p1 * float(KS - 1)
            f0 = v0 - v0.astype(jnp.int32).astype(jnp.float32)
            f1 = v1 - v1.astype(jnp.int32).astype(jnp.float32)
            b = [(f0 if (t & 1) else 1.0 - f0)
                 * (f1 if ((t >> 1) & 1) else 1.0 - f1) for t in range(KMAX)]
            for j in range(16):
                e = g * 16 + j
                bw = [jnp.full((16,), b[t][j], jnp.float32) for t in range(KMAX)]
                dsl = pl.ds(dst16[j], 16)
                deg_v[dsl] = deg_v[dsl] + onehot
                for col in range(D // 16):
                    sl = pl.ds(col * 16, 16)
                    slh = pl.ds(D + col * 16, 16)
                    a = rv[0, e, sl] * bw[0]
                    a = a + rv[0, e, slh] * bw[1]
                    a = a + rv[1, e, sl] * bw[2]
                    a = a + rv[1, e, slh] * bw[3]
                    av[e, sl] = a
            return ()
        lax.fori_loop(0, B1 // 16, _group, ())

    for c0 in range(NSL):
        fire_ed(c0, c0)
    for c0 in range(NSL - 1):
        wait_ed(c0, c0)
        phase_a(c0)
        fire_gathers(c0)

    def _iter(cio, _):
        for s in range(NSL):
            c = cio * NSL + s

            @pl.when(c + NSL - 1 < C1)
            def _():
                wait_ed(c + NSL - 1, (s + NSL - 1) % NSL)
                phase_a((s + NSL - 1) % NSL)
                fire_gathers((s + NSL - 1) % NSL)

            wait_gathers(s)

            @pl.when(c >= NSL)
            def _():
                wait_write(c - NSL, s)

            phase_b(s)
            fire_write(c, s)

            @pl.when(c + NSL < C1)
            def _():
                fire_ed(c + NSL, s)
        return ()

    lax.fori_loop(0, C1 // NSL, _iter, ())
    for c0 in range(NSL):
        wait_write(C1 - NSL + c0, c0 % NSL if False else (C1 - NSL + c0) % NSL)

    pltpu.sync_copy(deg_v.at[pl.ds(0, N_SH)], deg_hbm.at[wid])


_sc_msg = functools.partial(
    pl.kernel,
    mesh=plsc.VectorSubcoreMesh(core_axis_name="c", subcore_axis_name="s",
                                num_cores=NC),
    out_type=(jax.ShapeDtypeStruct((E_PAD, D), jnp.float32),
              jax.ShapeDtypeStruct((NW, N_SH), jnp.float32)),
    scratch_types=[
        *[pltpu.VMEM((2, B1), jnp.int32) for _ in range(NSL)],
        *[pltpu.VMEM((2, B1), jnp.float32) for _ in range(NSL)],
        *[pltpu.VMEM((NPAIR, B1), jnp.int32) for _ in range(NSL)],
        *[pltpu.VMEM((NPAIR, B1, TCOL), jnp.float32) for _ in range(NSL)],
        *[pltpu.VMEM((B1, D), jnp.float32) for _ in range(NSL)],
        pltpu.VMEM((N_SH + 16,), jnp.float32),
        *[pltpu.SemaphoreType.DMA for _ in range(4 * NSL)],
    ],
)(_sc_msg_body)


def _sc_scatter_body(dst_hbm, msg_hbm, zeros_hbm, out_hbm,
                     dst0, dst1, loc0, loc1, msg0, msg1, shared,
                     sd0, sd1, sm0, sm1, ss0, ss1):
    cid = lax.axis_index("c")
    sid = lax.axis_index("s")
    ebase = sid * EPC

    dstv = (dst0, dst1)
    locv = (loc0, loc1)
    msgv = (msg0, msg1)
    sd = (sd0, sd1)
    sm = (sm0, sm1)
    ss = (ss0, ss1)

    def fire_in(c, s):
        pltpu.async_copy(dst_hbm.at[pl.ds(ebase + c * B2, B2)], dstv[s].at[0],
                         sd[s])
        pltpu.async_copy(msg_hbm.at[pl.ds(ebase + c * B2, B2)], msgv[s], sm[s])

    def wait_in(c, s):
        pltpu.make_async_copy(dst_hbm.at[pl.ds(ebase + c * B2, B2)],
                              dstv[s].at[0], sd[s]).wait()
        pltpu.make_async_copy(msg_hbm.at[pl.ds(ebase + c * B2, B2)],
                              msgv[s], sm[s]).wait()

    for r in range(NRANGE // NC):
        q = r * NC + cid
        lo = q * RSZ

        pltpu.sync_copy(zeros_hbm, shared.at[pl.ds(sid * ZPS, ZPS)])
        plsc.subcore_barrier()

        fire_in(0, 0)

        def _iter(cio, _):
            for s in range(2):
                c = cio * 2 + s
                wait_in(c, s)
                for g in range(B2 // 16):
                    sl = pl.ds(g * 16, 16)
                    dg = dstv[s][0, sl]
                    inr = jnp.logical_and(dg >= lo, dg < lo + RSZ)
                    locv[s][0, sl] = jnp.where(inr, dg - lo, RSZ)

                @pl.when(c >= 1)
                def _():
                    pltpu.make_async_copy(msgv[1 - s],
                                          shared.at[locv[1 - s].at[0]],
                                          ss[1 - s]).wait()

                pltpu.async_copy(msgv[s], shared.at[locv[s].at[0]], ss[s],
                                 add=True)

                @pl.when(c + 1 < C2)
                def _():
                    fire_in(c + 1, 1 - s)
            return ()

        lax.fori_loop(0, C2 // 2, _iter, ())
        pltpu.make_async_copy(msgv[1], shared.at[locv[1].at[0]], ss[1]).wait()

        plsc.subcore_barrier()
        pltpu.sync_copy(shared.at[pl.ds(sid * CPS, CPS)],
                        out_hbm.at[q].at[pl.ds(sid * CPS, CPS)])
        plsc.subcore_barrier()


_sc_scatter = functools.partial(
    pl.kernel,
    mesh=plsc.VectorSubcoreMesh(core_axis_name="c", subcore_axis_name="s",
                                num_cores=NC),
    out_type=jax.ShapeDtypeStruct((NRANGE, RSZ, D), jnp.float32),
    scratch_types=[
        pltpu.VMEM((1, B2), jnp.int32),
        pltpu.VMEM((1, B2), jnp.int32),
        pltpu.VMEM((1, B2), jnp.int32),
        pltpu.VMEM((1, B2), jnp.int32),
        pltpu.VMEM((B2, D), jnp.float32),
        pltpu.VMEM((B2, D), jnp.float32),
        pltpu.VMEM_SHARED((ACC_ROWS, D), jnp.float32),
        pltpu.SemaphoreType.DMA,
        pltpu.SemaphoreType.DMA,
        pltpu.SemaphoreType.DMA,
        pltpu.SemaphoreType.DMA,
        pltpu.SemaphoreType.DMA,
        pltpu.SemaphoreType.DMA,
    ],
)(_sc_scatter_body)


def kernel(x, edge_index, pseudo, weight, bias):
    dst = edge_index[0]
    src = edge_index[1]
    pad = E_PAD - E
    dst_p = jnp.concatenate([dst, jnp.full((pad,), N, jnp.int32)])
    src_p = jnp.concatenate([src, jnp.zeros((pad,), jnp.int32)])
    p0 = jnp.concatenate([pseudo[:, 0], jnp.zeros((pad,), jnp.float32)])
    p1 = jnp.concatenate([pseudo[:, 1], jnp.zeros((pad,), jnp.float32)])
    ed = jnp.stack([dst_p, src_p])
    ed = ed.reshape(2, NW * C1, B1).transpose(1, 0, 2)
    pp = jnp.stack([p0, p1])
    pp = pp.reshape(2, NW * C1, B1).transpose(1, 0, 2)
    zeros = jnp.zeros((ZPS, D), jnp.float32)
    x_pad = jnp.concatenate([x, jnp.zeros((N_SH - N, D), jnp.float32)])

    ks = []
    for c1 in range(KS + 1):
        for c0 in range(KS):
            ks.append(c0 + KS * min(c1, KS - 1))
            ks.append(min(c0 + 1, KS - 1) + KS * min(c1, KS - 1))
    wt = jnp.transpose(weight[:K], (1, 0, 2))
    w2 = wt[:, jnp.array(ks, jnp.int32), :].reshape(D, TROW * TCOL)
    xw = pl.pallas_call(
        _xw_body,
        grid=(50,),
        in_specs=[pl.BlockSpec((200, D), lambda i: (i, 0)),
                  pl.BlockSpec((D, TROW * TCOL), lambda i: (0, 0))],
        out_specs=pl.BlockSpec((200, TROW * TCOL), lambda i: (i, 0)),
        out_shape=jax.ShapeDtypeStruct((N, TROW * TCOL), jnp.float32),
    )(x, w2)
    xw2d = xw.reshape(N * TROW, TCOL)

    msg, deg_parts = _sc_msg(ed, pp, xw2d)
    parts = _sc_scatter(dst_p, msg, zeros)
    msgsum = parts.reshape(NRANGE * RSZ, D)[:N_SH]

    out = pl.pallas_call(
        _finalize_body,
        grid=(N_SH // 128,),
        in_specs=[pl.BlockSpec((128, D), lambda i: (i, 0)),
                  pl.BlockSpec((NW, 128), lambda i: (0, i)),
                  pl.BlockSpec((128, D), lambda i: (i, 0)),
                  pl.BlockSpec((D, D), lambda i: (0, 0)),
                  pl.BlockSpec((1, D), lambda i: (0, 0))],
        out_specs=pl.BlockSpec((128, D), lambda i: (i, 0)),
        out_shape=jax.ShapeDtypeStruct((N_SH, D), jnp.float32),
    )(msgsum, deg_parts, x_pad, weight[K], bias.reshape(1, D))
    return out[:N]


# speedup vs baseline: 1.3354x; 1.3354x over previous
import functools

import jax
import jax.numpy as jnp
from jax import lax
from jax.experimental import pallas as pl
from jax.experimental.pallas import tpu as pltpu
from jax.experimental.pallas import tpu_sc as plsc

N = 10000
E = 320000
D = 128
K = 25
KS = 5
KMAX = 4

NC = 2
NS = 16
NW = NC * NS
E_PAD = 327680

B1 = 32
NSL = 5
EPW = E_PAD // NW
C1 = EPW // B1

B2 = 128
EPC = E_PAD // NS
C2 = EPC // B2

RSZ = 2560
NRANGE = 4
ACC_ROWS = RSZ + 128
ZPS = ACC_ROWS // NS
CPS = RSZ // NS
N_SH = 10112


def _xw_body(x_ref, w_ref, o_ref):
    o_ref[...] = jnp.dot(x_ref[...], w_ref[...], preferred_element_type=jnp.float32)


def _finalize_body(p_ref, dp_ref, x_ref, wr_ref, b_ref, o_ref):
    deg = jnp.sum(dp_ref[...], axis=0)[:, None]
    root = jnp.dot(x_ref[...], wr_ref[...], preferred_element_type=jnp.float32)
    o_ref[...] = p_ref[...] / jnp.maximum(deg, 1.0) + root + b_ref[...]


def _sc_msg_body(ed_hbm, pp_hbm, xw_hbm, msg_hbm, deg_hbm, *refs):
    cid = lax.axis_index("c")
    sid = lax.axis_index("s")
    wid = sid * NC + cid
    cbase = wid * C1

    ed = refs[0:NSL]
    pp = refs[NSL:2 * NSL]
    gi = refs[2 * NSL:3 * NSL]
    rows = refs[3 * NSL:4 * NSL]
    acc = refs[4 * NSL:5 * NSL]
    deg_v = refs[5 * NSL]
    sin = refs[5 * NSL + 1:5 * NSL + 1 + NSL]
    sp = refs[5 * NSL + 1 + NSL:5 * NSL + 1 + 2 * NSL]
    sg = refs[5 * NSL + 1 + 2 * NSL:5 * NSL + 1 + 3 * NSL]
    sw = refs[5 * NSL + 1 + 3 * NSL:5 * NSL + 1 + 4 * NSL]

    onehot = jnp.where(lax.iota(jnp.int32, 16) == 0, 1.0, 0.0).astype(jnp.float32)
    zeros16 = jnp.zeros((16,), jnp.float32)

    def _zero_deg(i, _):
        deg_v[pl.ds(i * 16, 16)] = zeros16
        return ()
    lax.fori_loop(0, (N_SH + 16) // 16, _zero_deg, ())

    def fire_ed(c, s):
        pltpu.async_copy(ed_hbm.at[cbase + c], ed[s], sin[s])
        pltpu.async_copy(pp_hbm.at[cbase + c], pp[s], sp[s])

    def wait_ed(c, s):
        pltpu.make_async_copy(ed_hbm.at[cbase + c], ed[s], sin[s]).wait()
        pltpu.make_async_copy(pp_hbm.at[cbase + c], pp[s], sp[s]).wait()

    def phase_a(s):
        edv = ed[s]
        ppv = pp[s]
        giv = gi[s]
        for g in range(B1 // 16):
            sl = pl.ds(g * 16, 16)
            srcv = edv[1, sl]
            p0 = ppv[0, sl]
            p1 = ppv[1, sl]
            b0i = (p0 * float(KS - 1)).astype(jnp.int32)
            b1i = (p1 * float(KS - 1)).astype(jnp.int32)
            for t in range(KMAX):
                bit0 = t & 1
                bit1 = (t >> 1) & 1
                wi = (jnp.minimum(b0i + bit0, KS - 1)
                      + KS * jnp.minimum(b1i + bit1, KS - 1))
                giv[t, sl] = srcv * K + wi

    def fire_gathers(s):
        for t in range(KMAX):
            pltpu.async_copy(xw_hbm.at[gi[s].at[t]], rows[s].at[t], sg[s])

    def wait_gathers(s):
        for t in range(KMAX):
            pltpu.make_async_copy(xw_hbm.at[gi[s].at[t]], rows[s].at[t],
                                  sg[s]).wait()

    def fire_write(c, s):
        pltpu.async_copy(acc[s], msg_hbm.at[pl.ds((cbase + c) * B1, B1)], sw[s])

    def wait_write(c, s):
        pltpu.make_async_copy(acc[s], msg_hbm.at[pl.ds((cbase + c) * B1, B1)],
                              sw[s]).wait()

    def phase_b(s):
        edv = ed[s]
        ppv = pp[s]
        rv = rows[s]
        av = acc[s]

        def _group(g, _):
            sl16 = pl.ds(g * 16, 16)
            dst16 = edv[0, sl16]
            p0 = ppv[0, sl16]
            p1 = ppv[1, sl16]
            v0 = p0 * float(KS - 1)
            v1 = p1 * float(KS - 1)
            f0 = v0 - v0.astype(jnp.int32).astype(jnp.float32)
            f1 = v1 - v1.astype(jnp.int32).astype(jnp.float32)
            b = [(f0 if (t & 1) else 1.0 - f0)
                 * (f1 if ((t >> 1) & 1) else 1.0 - f1) for t in range(KMAX)]
            for j in range(16):
                e = g * 16 + j
                bw = [jnp.full((16,), b[t][j], jnp.float32) for t in range(KMAX)]
                dsl = pl.ds(dst16[j], 16)
                deg_v[dsl] = deg_v[dsl] + onehot
                for col in range(D // 16):
                    sl = pl.ds(col * 16, 16)
                    a = rv[0, e, sl] * bw[0]
                    a = a + rv[1, e, sl] * bw[1]
                    a = a + rv[2, e, sl] * bw[2]
                    a = a + rv[3, e, sl] * bw[3]
                    av[e, sl] = a
            return ()
        lax.fori_loop(0, B1 // 16, _group, ())

    for c0 in range(NSL):
        fire_ed(c0, c0)
    for c0 in range(NSL - 1):
        wait_ed(c0, c0)
        phase_a(c0)
        fire_gathers(c0)

    def _iter(cio, _):
        for s in range(NSL):
            c = cio * NSL + s

            @pl.when(c + NSL - 1 < C1)
            def _():
                wait_ed(c + NSL - 1, (s + NSL - 1) % NSL)
                phase_a((s + NSL - 1) % NSL)
                fire_gathers((s + NSL - 1) % NSL)

            wait_gathers(s)

            @pl.when(c >= NSL)
            def _():
                wait_write(c - NSL, s)

            phase_b(s)
            fire_write(c, s)

            @pl.when(c + NSL < C1)
            def _():
                fire_ed(c + NSL, s)
        return ()

    lax.fori_loop(0, C1 // NSL, _iter, ())
    for c0 in range(NSL):
        wait_write(C1 - NSL + c0, c0 % NSL if False else (C1 - NSL + c0) % NSL)

    pltpu.sync_copy(deg_v.at[pl.ds(0, N_SH)], deg_hbm.at[wid])


_sc_msg = functools.partial(
    pl.kernel,
    mesh=plsc.VectorSubcoreMesh(core_axis_name="c", subcore_axis_name="s",
                                num_cores=NC),
    out_type=(jax.ShapeDtypeStruct((E_PAD, D), jnp.float32),
              jax.ShapeDtypeStruct((NW, N_SH), jnp.float32)),
    scratch_types=[
        *[pltpu.VMEM((2, B1), jnp.int32) for _ in range(NSL)],
        *[pltpu.VMEM((2, B1), jnp.float32) for _ in range(NSL)],
        *[pltpu.VMEM((KMAX, B1), jnp.int32) for _ in range(NSL)],
        *[pltpu.VMEM((KMAX, B1, D), jnp.float32) for _ in range(NSL)],
        *[pltpu.VMEM((B1, D), jnp.float32) for _ in range(NSL)],
        pltpu.VMEM((N_SH + 16,), jnp.float32),
        *[pltpu.SemaphoreType.DMA for _ in range(4 * NSL)],
    ],
)(_sc_msg_body)


def _sc_scatter_body(dst_hbm, msg_hbm, zeros_hbm, out_hbm,
                     dst0, dst1, loc0, loc1, msg0, msg1, shared,
                     sd0, sd1, sm0, sm1, ss0, ss1):
    cid = lax.axis_index("c")
    sid = lax.axis_index("s")
    ebase = sid * EPC

    dstv = (dst0, dst1)
    locv = (loc0, loc1)
    msgv = (msg0, msg1)
    sd = (sd0, sd1)
    sm = (sm0, sm1)
    ss = (ss0, ss1)

    def fire_in(c, s):
        pltpu.async_copy(dst_hbm.at[pl.ds(ebase + c * B2, B2)], dstv[s].at[0],
                         sd[s])
        pltpu.async_copy(msg_hbm.at[pl.ds(ebase + c * B2, B2)], msgv[s], sm[s])

    def wait_in(c, s):
        pltpu.make_async_copy(dst_hbm.at[pl.ds(ebase + c * B2, B2)],
                              dstv[s].at[0], sd[s]).wait()
        pltpu.make_async_copy(msg_hbm.at[pl.ds(ebase + c * B2, B2)],
                              msgv[s], sm[s]).wait()

    for r in range(NRANGE // NC):
        q = r * NC + cid
        lo = q * RSZ

        pltpu.sync_copy(zeros_hbm, shared.at[pl.ds(sid * ZPS, ZPS)])
        plsc.subcore_barrier()

        fire_in(0, 0)

        def _iter(cio, _):
            for s in range(2):
                c = cio * 2 + s
                wait_in(c, s)
                for g in range(B2 // 16):
                    sl = pl.ds(g * 16, 16)
                    dg = dstv[s][0, sl]
                    inr = jnp.logical_and(dg >= lo, dg < lo + RSZ)
                    locv[s][0, sl] = jnp.where(inr, dg - lo, RSZ)

                @pl.when(c >= 1)
                def _():
                    pltpu.make_async_copy(msgv[1 - s],
                                          shared.at[locv[1 - s].at[0]],
                                          ss[1 - s]).wait()

                pltpu.async_copy(msgv[s], shared.at[locv[s].at[0]], ss[s],
                                 add=True)

                @pl.when(c + 1 < C2)
                def _():
                    fire_in(c + 1, 1 - s)
            return ()

        lax.fori_loop(0, C2 // 2, _iter, ())
        pltpu.make_async_copy(msgv[1], shared.at[locv[1].at[0]], ss[1]).wait()

        plsc.subcore_barrier()
        pltpu.sync_copy(shared.at[pl.ds(sid * CPS, CPS)],
                        out_hbm.at[q].at[pl.ds(sid * CPS, CPS)])
        plsc.subcore_barrier()


_sc_scatter = functools.partial(
    pl.kernel,
    mesh=plsc.VectorSubcoreMesh(core_axis_name="c", subcore_axis_name="s",
                                num_cores=NC),
    out_type=jax.ShapeDtypeStruct((NRANGE, RSZ, D), jnp.float32),
    scratch_types=[
        pltpu.VMEM((1, B2), jnp.int32),
        pltpu.VMEM((1, B2), jnp.int32),
        pltpu.VMEM((1, B2), jnp.int32),
        pltpu.VMEM((1, B2), jnp.int32),
        pltpu.VMEM((B2, D), jnp.float32),
        pltpu.VMEM((B2, D), jnp.float32),
        pltpu.VMEM_SHARED((ACC_ROWS, D), jnp.float32),
        pltpu.SemaphoreType.DMA,
        pltpu.SemaphoreType.DMA,
        pltpu.SemaphoreType.DMA,
        pltpu.SemaphoreType.DMA,
        pltpu.SemaphoreType.DMA,
        pltpu.SemaphoreType.DMA,
    ],
)(_sc_scatter_body)


def kernel(x, edge_index, pseudo, weight, bias):
    dst = edge_index[0]
    src = edge_index[1]
    pad = E_PAD - E
    dst_p = jnp.concatenate([dst, jnp.full((pad,), N, jnp.int32)])
    src_p = jnp.concatenate([src, jnp.zeros((pad,), jnp.int32)])
    p0 = jnp.concatenate([pseudo[:, 0], jnp.zeros((pad,), jnp.float32)])
    p1 = jnp.concatenate([pseudo[:, 1], jnp.zeros((pad,), jnp.float32)])
    ed = jnp.stack([dst_p, src_p])
    ed = ed.reshape(2, NW * C1, B1).transpose(1, 0, 2)
    pp = jnp.stack([p0, p1])
    pp = pp.reshape(2, NW * C1, B1).transpose(1, 0, 2)
    zeros = jnp.zeros((ZPS, D), jnp.float32)
    x_pad = jnp.concatenate([x, jnp.zeros((N_SH - N, D), jnp.float32)])

    w2 = jnp.transpose(weight[:K], (1, 0, 2)).reshape(D, K * D)
    xw = pl.pallas_call(
        _xw_body,
        grid=(25,),
        in_specs=[pl.BlockSpec((400, D), lambda i: (i, 0)),
                  pl.BlockSpec((D, K * D), lambda i: (0, 0))],
        out_specs=pl.BlockSpec((400, K * D), lambda i: (i, 0)),
        out_shape=jax.ShapeDtypeStruct((N, K * D), jnp.float32),
    )(x, w2)
    xw2d = xw.reshape(N * K, D)

    msg, deg_parts = _sc_msg(ed, pp, xw2d)
    parts = _sc_scatter(dst_p, msg, zeros)
    msgsum = parts.reshape(NRANGE * RSZ, D)[:N_SH]

    out = pl.pallas_call(
        _finalize_body,
        grid=(N_SH // 128,),
        in_specs=[pl.BlockSpec((128, D), lambda i: (i, 0)),
                  pl.BlockSpec((NW, 128), lambda i: (0, i)),
                  pl.BlockSpec((128, D), lambda i: (i, 0)),
                  pl.BlockSpec((D, D), lambda i: (0, 0)),
                  pl.BlockSpec((1, D), lambda i: (0, 0))],
        out_specs=pl.BlockSpec((128, D), lambda i: (i, 0)),
        out_shape=jax.ShapeDtypeStruct((N_SH, D), jnp.float32),
    )(msgsum, deg_parts, x_pad, weight[K], bias.reshape(1, D))
    return out[:N]


# speedup vs baseline: 1.4008x; 1.0490x over previous
import functools

import jax
import jax.numpy as jnp
from jax import lax
from jax.experimental import pallas as pl
from jax.experimental.pallas import tpu as pltpu
from jax.experimental.pallas import tpu_sc as plsc

N = 10000
E = 320000
D = 128
K = 25
KS = 5
KMAX = 4

NC = 2
NS = 16
NW = NC * NS
E_PAD = 323584

B1 = 32
NSL = 4
EPW = E_PAD // NW
C1 = EPW // B1

B2 = 128
EPC = E_PAD // NS
C2 = EPC // B2

RSZ = 2560
NRANGE = 4
ACC_ROWS = RSZ + 128
ZPS = ACC_ROWS // NS
CPS = RSZ // NS
N_SH = 10112


def _xw_body(x_ref, w_ref, o_ref):
    o_ref[...] = jnp.dot(x_ref[...], w_ref[...], preferred_element_type=jnp.float32)


def _finalize_body(p_ref, dp_ref, x_ref, wr_ref, b_ref, o_ref):
    deg = jnp.sum(dp_ref[...], axis=0)[:, None]
    root = jnp.dot(x_ref[...], wr_ref[...], preferred_element_type=jnp.float32)
    o_ref[...] = p_ref[...] / jnp.maximum(deg, 1.0) + root + b_ref[...]


def _sc_msg_body(ed_hbm, pp_hbm, xw_hbm, msg_hbm, deg_hbm, *refs):
    cid = lax.axis_index("c")
    sid = lax.axis_index("s")
    wid = sid * NC + cid
    cbase = wid * C1

    ed = refs[0:NSL]
    pp = refs[NSL:2 * NSL]
    gi = refs[2 * NSL:3 * NSL]
    rows = refs[3 * NSL:4 * NSL]
    acc = refs[4 * NSL:5 * NSL]
    deg_v = refs[5 * NSL]
    sin = refs[5 * NSL + 1:5 * NSL + 1 + NSL]
    sp = refs[5 * NSL + 1 + NSL:5 * NSL + 1 + 2 * NSL]
    sg = refs[5 * NSL + 1 + 2 * NSL:5 * NSL + 1 + 3 * NSL]
    sw = refs[5 * NSL + 1 + 3 * NSL:5 * NSL + 1 + 4 * NSL]

    onehot = jnp.where(lax.iota(jnp.int32, 16) == 0, 1.0, 0.0).astype(jnp.float32)
    zeros16 = jnp.zeros((16,), jnp.float32)

    def _zero_deg(i, _):
        deg_v[pl.ds(i * 16, 16)] = zeros16
        return ()
    lax.fori_loop(0, (N_SH + 16) // 16, _zero_deg, ())

    def fire_ed(c, s):
        pltpu.async_copy(ed_hbm.at[cbase + c], ed[s], sin[s])
        pltpu.async_copy(pp_hbm.at[cbase + c], pp[s], sp[s])

    def wait_ed(c, s):
        pltpu.make_async_copy(ed_hbm.at[cbase + c], ed[s], sin[s]).wait()
        pltpu.make_async_copy(pp_hbm.at[cbase + c], pp[s], sp[s]).wait()

    def phase_a(s):
        edv = ed[s]
        ppv = pp[s]
        giv = gi[s]
        for g in range(B1 // 16):
            sl = pl.ds(g * 16, 16)
            srcv = edv[1, sl]
            p0 = ppv[0, sl]
            p1 = ppv[1, sl]
            b0i = (p0 * float(KS - 1)).astype(jnp.int32)
            b1i = (p1 * float(KS - 1)).astype(jnp.int32)
            for t in range(KMAX):
                bit0 = t & 1
                bit1 = (t >> 1) & 1
                wi = (jnp.minimum(b0i + bit0, KS - 1)
                      + KS * jnp.minimum(b1i + bit1, KS - 1))
                giv[t, sl] = srcv * K + wi

    def fire_gathers(s):
        for t in range(KMAX):
            pltpu.async_copy(xw_hbm.at[gi[s].at[t]], rows[s].at[t], sg[s])

    def wait_gathers(s):
        for t in range(KMAX):
            pltpu.make_async_copy(xw_hbm.at[gi[s].at[t]], rows[s].at[t],
                                  sg[s]).wait()

    def fire_write(c, s):
        pltpu.async_copy(acc[s], msg_hbm.at[pl.ds((cbase + c) * B1, B1)], sw[s])

    def wait_write(c, s):
        pltpu.make_async_copy(acc[s], msg_hbm.at[pl.ds((cbase + c) * B1, B1)],
                              sw[s]).wait()

    def phase_b(s):
        edv = ed[s]
        ppv = pp[s]
        rv = rows[s]
        av = acc[s]

        def _group(g, _):
            sl16 = pl.ds(g * 16, 16)
            dst16 = edv[0, sl16]
            p0 = ppv[0, sl16]
            p1 = ppv[1, sl16]
            v0 = p0 * float(KS - 1)
            v1 = p1 * float(KS - 1)
            f0 = v0 - v0.astype(jnp.int32).astype(jnp.float32)
            f1 = v1 - v1.astype(jnp.int32).astype(jnp.float32)
            b = [(f0 if (t & 1) else 1.0 - f0)
                 * (f1 if ((t >> 1) & 1) else 1.0 - f1) for t in range(KMAX)]
            for j in range(16):
                e = g * 16 + j
                bw = [jnp.full((16,), b[t][j], jnp.float32) for t in range(KMAX)]
                dsl = pl.ds(dst16[j], 16)
                deg_v[dsl] = deg_v[dsl] + onehot
                for col in range(D // 16):
                    sl = pl.ds(col * 16, 16)
                    a = rv[0, e, sl] * bw[0]
                    a = a + rv[1, e, sl] * bw[1]
                    a = a + rv[2, e, sl] * bw[2]
                    a = a + rv[3, e, sl] * bw[3]
                    av[e, sl] = a
            return ()
        lax.fori_loop(0, B1 // 16, _group, ())

    for c0 in range(NSL):
        fire_ed(c0, c0)
    for c0 in range(NSL - 1):
        wait_ed(c0, c0)
        phase_a(c0)
        fire_gathers(c0)

    def _iter(cio, _):
        for s in range(NSL):
            c = cio * NSL + s

            @pl.when(c + NSL - 1 < C1)
            def _():
                wait_ed(c + NSL - 1, (s + NSL - 1) % NSL)
                phase_a((s + NSL - 1) % NSL)
                fire_gathers((s + NSL - 1) % NSL)

            wait_gathers(s)

            @pl.when(c >= NSL)
            def _():
                wait_write(c - NSL, s)

            phase_b(s)
            fire_write(c, s)

            @pl.when(c + NSL < C1)
            def _():
                fire_ed(c + NSL, s)
        return ()

    lax.fori_loop(0, C1 // NSL, _iter, ())
    for c0 in range(NSL):
        wait_write(C1 - NSL + c0, c0 % NSL if False else (C1 - NSL + c0) % NSL)

    pltpu.sync_copy(deg_v.at[pl.ds(0, N_SH)], deg_hbm.at[wid])


_sc_msg = functools.partial(
    pl.kernel,
    mesh=plsc.VectorSubcoreMesh(core_axis_name="c", subcore_axis_name="s",
                                num_cores=NC),
    out_type=(jax.ShapeDtypeStruct((E_PAD, D), jnp.float32),
              jax.ShapeDtypeStruct((NW, N_SH), jnp.float32)),
    scratch_types=[
        *[pltpu.VMEM((2, B1), jnp.int32) for _ in range(NSL)],
        *[pltpu.VMEM((2, B1), jnp.float32) for _ in range(NSL)],
        *[pltpu.VMEM((KMAX, B1), jnp.int32) for _ in range(NSL)],
        *[pltpu.VMEM((KMAX, B1, D), jnp.float32) for _ in range(NSL)],
        *[pltpu.VMEM((B1, D), jnp.float32) for _ in range(NSL)],
        pltpu.VMEM((N_SH + 16,), jnp.float32),
        *[pltpu.SemaphoreType.DMA for _ in range(4 * NSL)],
    ],
)(_sc_msg_body)


def _sc_scatter_body(dst_hbm, msg_hbm, zeros_hbm, out_hbm,
                     dst0, dst1, loc0, loc1, msg0, msg1, shared,
                     sd0, sd1, sm0, sm1, ss0, ss1):
    cid = lax.axis_index("c")
    sid = lax.axis_index("s")
    ebase = sid * EPC

    dstv = (dst0, dst1)
    locv = (loc0, loc1)
    msgv = (msg0, msg1)
    sd = (sd0, sd1)
    sm = (sm0, sm1)
    ss = (ss0, ss1)

    def fire_in(c, s):
        pltpu.async_copy(dst_hbm.at[pl.ds(ebase + c * B2, B2)], dstv[s].at[0],
                         sd[s])
        pltpu.async_copy(msg_hbm.at[pl.ds(ebase + c * B2, B2)], msgv[s], sm[s])

    def wait_in(c, s):
        pltpu.make_async_copy(dst_hbm.at[pl.ds(ebase + c * B2, B2)],
                              dstv[s].at[0], sd[s]).wait()
        pltpu.make_async_copy(msg_hbm.at[pl.ds(ebase + c * B2, B2)],
                              msgv[s], sm[s]).wait()

    for r in range(NRANGE // NC):
        q = r * NC + cid
        lo = q * RSZ

        pltpu.sync_copy(zeros_hbm, shared.at[pl.ds(sid * ZPS, ZPS)])
        plsc.subcore_barrier()

        fire_in(0, 0)

        def _iter(cio, _):
            for s in range(2):
                c = cio * 2 + s
                wait_in(c, s)
                for g in range(B2 // 16):
                    sl = pl.ds(g * 16, 16)
                    dg = dstv[s][0, sl]
                    inr = jnp.logical_and(dg >= lo, dg < lo + RSZ)
                    locv[s][0, sl] = jnp.where(inr, dg - lo, RSZ)

                @pl.when(c >= 1)
                def _():
                    pltpu.make_async_copy(msgv[1 - s],
                                          shared.at[locv[1 - s].at[0]],
                                          ss[1 - s]).wait()

                pltpu.async_copy(msgv[s], shared.at[locv[s].at[0]], ss[s],
                                 add=True)

                @pl.when(c + 1 < C2)
                def _():
                    fire_in(c + 1, 1 - s)
            return ()

        lax.fori_loop(0, C2 // 2, _iter, ())
        pltpu.make_async_copy(msgv[1], shared.at[locv[1].at[0]], ss[1]).wait()

        plsc.subcore_barrier()
        pltpu.sync_copy(shared.at[pl.ds(sid * CPS, CPS)],
                        out_hbm.at[q].at[pl.ds(sid * CPS, CPS)])
        plsc.subcore_barrier()


_sc_scatter = functools.partial(
    pl.kernel,
    mesh=plsc.VectorSubcoreMesh(core_axis_name="c", subcore_axis_name="s",
                                num_cores=NC),
    out_type=jax.ShapeDtypeStruct((NRANGE, RSZ, D), jnp.float32),
    scratch_types=[
        pltpu.VMEM((1, B2), jnp.int32),
        pltpu.VMEM((1, B2), jnp.int32),
        pltpu.VMEM((1, B2), jnp.int32),
        pltpu.VMEM((1, B2), jnp.int32),
        pltpu.VMEM((B2, D), jnp.float32),
        pltpu.VMEM((B2, D), jnp.float32),
        pltpu.VMEM_SHARED((ACC_ROWS, D), jnp.float32),
        pltpu.SemaphoreType.DMA,
        pltpu.SemaphoreType.DMA,
        pltpu.SemaphoreType.DMA,
        pltpu.SemaphoreType.DMA,
        pltpu.SemaphoreType.DMA,
        pltpu.SemaphoreType.DMA,
    ],
)(_sc_scatter_body)


def kernel(x, edge_index, pseudo, weight, bias):
    dst = edge_index[0]
    src = edge_index[1]
    pad = E_PAD - E
    dst_p = jnp.concatenate([dst, jnp.full((pad,), N, jnp.int32)])
    src_p = jnp.concatenate([src, jnp.zeros((pad,), jnp.int32)])
    p0 = jnp.concatenate([pseudo[:, 0], jnp.zeros((pad,), jnp.float32)])
    p1 = jnp.concatenate([pseudo[:, 1], jnp.zeros((pad,), jnp.float32)])
    ed = jnp.stack([dst_p, src_p])
    ed = ed.reshape(2, NW * C1, B1).transpose(1, 0, 2)
    pp = jnp.stack([p0, p1])
    pp = pp.reshape(2, NW * C1, B1).transpose(1, 0, 2)
    zeros = jnp.zeros((ZPS, D), jnp.float32)
    x_pad = jnp.concatenate([x, jnp.zeros((N_SH - N, D), jnp.float32)])

    w2 = jnp.transpose(weight[:K], (1, 0, 2)).reshape(D, K * D)
    xw = pl.pallas_call(
        _xw_body,
        grid=(25,),
        in_specs=[pl.BlockSpec((400, D), lambda i: (i, 0)),
                  pl.BlockSpec((D, K * D), lambda i: (0, 0))],
        out_specs=pl.BlockSpec((400, K * D), lambda i: (i, 0)),
        out_shape=jax.ShapeDtypeStruct((N, K * D), jnp.float32),
    )(x, w2)
    xw2d = xw.reshape(N * K, D)

    msg, deg_parts = _sc_msg(ed, pp, xw2d)
    parts = _sc_scatter(dst_p, msg, zeros)
    msgsum = parts.reshape(NRANGE * RSZ, D)[:N_SH]

    out = pl.pallas_call(
        _finalize_body,
        grid=(N_SH // 128,),
        in_specs=[pl.BlockSpec((128, D), lambda i: (i, 0)),
                  pl.BlockSpec((NW, 128), lambda i: (0, i)),
                  pl.BlockSpec((128, D), lambda i: (i, 0)),
                  pl.BlockSpec((D, D), lambda i: (0, 0)),
                  pl.BlockSpec((1, D), lambda i: (0, 0))],
        out_specs=pl.BlockSpec((128, D), lambda i: (i, 0)),
        out_shape=jax.ShapeDtypeStruct((N_SH, D), jnp.float32),
    )(msgsum, deg_parts, x_pad, weight[K], bias.reshape(1, D))
    return out[:N]
